# Initial kernel scaffold; baseline (speedup 1.0000x reference)
import jax
import jax.numpy as jnp
from jax.experimental import pallas as pl


def kernel(x, f_ij, idx_i, idx_j, rcut_ij, W_in, b_in, W_filt, b_filt, W_out, b_out):
    raise NotImplementedError("write your pallas kernel here")


# speedup vs baseline: 2.3294x; 2.3294x over previous
import functools

import jax
import jax.numpy as jnp
from jax import lax
from jax.experimental import pallas as pl
from jax.experimental.pallas import tpu as pltpu
from jax.experimental.pallas import tpu_sc as plsc

_LOG2 = 0.6931471805599453

_N_ATOMS = 10000
_N_PAIRS = 320000
_NF = 128

_NC = 2
_NS = 16
_NW = _NC * _NS
_C = 128
_NCHUNK = _N_PAIRS // _C
_ROWS_PER_TILE = 624


def _shifted_softplus(t):
    return jnp.maximum(t, 0.0) + jnp.log1p(jnp.exp(-jnp.abs(t))) - _LOG2


def _h_body(x_ref, w_ref, b_ref, o_ref):
    o_ref[...] = (
        jnp.dot(x_ref[...], w_ref[...], preferred_element_type=jnp.float32)
        + b_ref[...]
    )


def _compute_h(x2d, w_in_t, b_in2d):
    blk = 2000
    grid = _N_ATOMS // blk
    return pl.pallas_call(
        _h_body,
        grid=(grid,),
        in_specs=[
            pl.BlockSpec((blk, _NF), lambda i: (i, 0)),
            pl.BlockSpec((_NF, _NF), lambda i: (0, 0)),
            pl.BlockSpec((1, _NF), lambda i: (0, 0)),
        ],
        out_specs=pl.BlockSpec((blk, _NF), lambda i: (i, 0)),
        out_shape=jax.ShapeDtypeStruct((_N_ATOMS, _NF), jnp.float32),
    )(x2d, w_in_t, b_in2d)


def _wij_body(f_ref, rc_ref, w_ref, b_ref, o_ref):
    t = (
        jnp.dot(f_ref[...], w_ref[...], preferred_element_type=jnp.float32)
        + b_ref[...]
    )
    o_ref[...] = _shifted_softplus(t) * rc_ref[...]


def _compute_wij(f_ij, rcut2d, w_filt_t, b_filt2d):
    blk = 4000
    grid = _N_PAIRS // blk
    n_rbf = f_ij.shape[1]
    return pl.pallas_call(
        _wij_body,
        grid=(grid,),
        in_specs=[
            pl.BlockSpec((blk, n_rbf), lambda i: (i, 0)),
            pl.BlockSpec((blk, 1), lambda i: (i, 0)),
            pl.BlockSpec((n_rbf, _NF), lambda i: (0, 0)),
            pl.BlockSpec((1, _NF), lambda i: (0, 0)),
        ],
        out_specs=pl.BlockSpec((blk, _NF), lambda i: (i, 0)),
        out_shape=jax.ShapeDtypeStruct((_N_PAIRS, _NF), jnp.float32),
    )(f_ij, rcut2d, w_filt_t, b_filt2d)


def _sc_edge_body(h_hbm, wij_hbm, idxi_hbm, idxj_hbm, out_hbm,
                  idxi_v, idxj_v, wij_v, rows_v, zero_v, agg_sh, sem):
    cid = lax.axis_index("c")
    sid = lax.axis_index("s")
    wid = cid * _NS + sid

    z16 = jnp.zeros((16,), jnp.float32)

    def _zb(i, carry):
        r = i // 8
        c = (i % 8) * 16
        zero_v[r, pl.ds(c, 16)] = z16
        return carry

    lax.fori_loop(0, 128 * 8, _zb, 0)
    base_rows = sid * _ROWS_PER_TILE
    for k in range(4):
        pltpu.sync_copy(zero_v, agg_sh.at[pl.ds(base_rows + k * 128, 128)])
    pltpu.sync_copy(zero_v.at[pl.ds(0, _ROWS_PER_TILE - 512)],
                    agg_sh.at[pl.ds(base_rows + 512, _ROWS_PER_TILE - 512)])

    @pl.when(sid == _NS - 1)
    def _zero_tail():
        pltpu.sync_copy(zero_v.at[pl.ds(0, _N_ATOMS - _NS * _ROWS_PER_TILE)],
                        agg_sh.at[pl.ds(_NS * _ROWS_PER_TILE,
                                        _N_ATOMS - _NS * _ROWS_PER_TILE)])

    plsc.subcore_barrier()

    nmine = _NCHUNK // _NW + jnp.where(wid < _NCHUNK % _NW, 1, 0)

    def _chunk(t, carry):
        base = (wid + t * _NW) * _C
        pltpu.sync_copy(idxj_hbm.at[pl.ds(base, _C)], idxj_v)
        pltpu.sync_copy(idxi_hbm.at[pl.ds(base, _C)], idxi_v)
        gcp = pltpu.async_copy(h_hbm.at[idxj_v], rows_v, sem)
        pltpu.sync_copy(wij_hbm.at[pl.ds(base, _C)], wij_v)
        gcp.wait()

        def _mrow(i, c2):
            for l in range(8):
                s = pl.ds(l * 16, 16)
                rows_v[i, s] = rows_v[i, s] * wij_v[i, s]
            return c2

        lax.fori_loop(0, _C, _mrow, 0)
        pltpu.sync_copy(rows_v, agg_sh.at[idxi_v], add=True)
        return carry

    lax.fori_loop(0, nmine, _chunk, 0)
    plsc.subcore_barrier()

    pltpu.sync_copy(agg_sh.at[pl.ds(base_rows, _ROWS_PER_TILE)],
                    out_hbm.at[cid, pl.ds(base_rows, _ROWS_PER_TILE)])

    @pl.when(sid == _NS - 1)
    def _write_tail():
        tail = _N_ATOMS - _NS * _ROWS_PER_TILE
        pltpu.sync_copy(agg_sh.at[pl.ds(_NS * _ROWS_PER_TILE, tail)],
                        out_hbm.at[cid, pl.ds(_NS * _ROWS_PER_TILE, tail)])


def _sc_edge(h, wij, idx_i, idx_j):
    mesh = plsc.VectorSubcoreMesh(core_axis_name="c", subcore_axis_name="s")
    f = functools.partial(
        pl.kernel,
        mesh=mesh,
        out_type=jax.ShapeDtypeStruct((_NC, _N_ATOMS, _NF), jnp.float32),
        scratch_types=[
            pltpu.VMEM((_C,), jnp.int32),
            pltpu.VMEM((_C,), jnp.int32),
            pltpu.VMEM((_C, _NF), jnp.float32),
            pltpu.VMEM((_C, _NF), jnp.float32),
            pltpu.VMEM((128, _NF), jnp.float32),
            pltpu.VMEM_SHARED((_N_ATOMS, _NF), jnp.float32),
            pltpu.SemaphoreType.DMA,
        ],
    )(_sc_edge_body)
    return f(h, wij, idx_i, idx_j)


def _out_body(p_ref, w_ref, b_ref, o_ref):
    a = p_ref[0] + p_ref[1]
    t = jnp.dot(a, w_ref[...], preferred_element_type=jnp.float32) + b_ref[...]
    o_ref[...] = _shifted_softplus(t)


def _compute_out(partials, w_out_t, b_out2d):
    blk = 2000
    grid = _N_ATOMS // blk
    return pl.pallas_call(
        _out_body,
        grid=(grid,),
        in_specs=[
            pl.BlockSpec((2, blk, _NF), lambda i: (0, i, 0)),
            pl.BlockSpec((_NF, _NF), lambda i: (0, 0)),
            pl.BlockSpec((1, _NF), lambda i: (0, 0)),
        ],
        out_specs=pl.BlockSpec((blk, _NF), lambda i: (i, 0)),
        out_shape=jax.ShapeDtypeStruct((_N_ATOMS, _NF), jnp.float32),
    )(partials, w_out_t, b_out2d)


def kernel(x, f_ij, idx_i, idx_j, rcut_ij, W_in, b_in, W_filt, b_filt,
           W_out, b_out):
    batch, n_atoms = x.shape[0], x.shape[1]
    x2d = x.reshape(batch * n_atoms, _NF)
    h = _compute_h(x2d, W_in.T, b_in.reshape(1, _NF))
    wij = _compute_wij(f_ij, rcut_ij.reshape(_N_PAIRS, 1), W_filt.T,
                       b_filt.reshape(1, _NF))
    partials = _sc_edge(h, wij, idx_i.astype(jnp.int32),
                        idx_j.astype(jnp.int32))
    out = _compute_out(partials, W_out.T, b_out.reshape(1, _NF))
    return out.reshape(batch, n_atoms, _NF)


# speedup vs baseline: 3.0327x; 1.3020x over previous
import functools

import jax
import jax.numpy as jnp
from jax import lax
from jax.experimental import pallas as pl
from jax.experimental.pallas import tpu as pltpu
from jax.experimental.pallas import tpu_sc as plsc

_LOG2 = 0.6931471805599453

_N_ATOMS = 10000
_N_PAIRS = 320000
_NF = 128

_NC = 2
_NS = 16
_NW = _NC * _NS
_C = 64
_NCHUNK = _N_PAIRS // _C
_CPW = _NCHUNK // _NW
_NTAIL = _NCHUNK - _CPW * _NW
_U = 12
_ROWS_PER_TILE = 624


def _shifted_softplus(t):
    return jnp.maximum(t, 0.0) + jnp.log1p(jnp.exp(-jnp.abs(t))) - _LOG2


def _h_body(x_ref, w_ref, b_ref, o_ref):
    o_ref[...] = (
        jnp.dot(x_ref[...], w_ref[...], preferred_element_type=jnp.float32)
        + b_ref[...]
    )


def _compute_h(x2d, w_in_t, b_in2d):
    blk = 2000
    grid = _N_ATOMS // blk
    return pl.pallas_call(
        _h_body,
        grid=(grid,),
        in_specs=[
            pl.BlockSpec((blk, _NF), lambda i: (i, 0)),
            pl.BlockSpec((_NF, _NF), lambda i: (0, 0)),
            pl.BlockSpec((1, _NF), lambda i: (0, 0)),
        ],
        out_specs=pl.BlockSpec((blk, _NF), lambda i: (i, 0)),
        out_shape=jax.ShapeDtypeStruct((_N_ATOMS, _NF), jnp.float32),
    )(x2d, w_in_t, b_in2d)


def _wij_body(f_ref, rc_ref, w_ref, b_ref, o_ref):
    t = (
        jnp.dot(f_ref[...], w_ref[...], preferred_element_type=jnp.float32)
        + b_ref[...]
    )
    o_ref[...] = _shifted_softplus(t) * rc_ref[...]


def _compute_wij(f_ij, rcut2d, w_filt_t, b_filt2d):
    blk = 4000
    grid = _N_PAIRS // blk
    n_rbf = f_ij.shape[1]
    return pl.pallas_call(
        _wij_body,
        grid=(grid,),
        in_specs=[
            pl.BlockSpec((blk, n_rbf), lambda i: (i, 0)),
            pl.BlockSpec((blk, 1), lambda i: (i, 0)),
            pl.BlockSpec((n_rbf, _NF), lambda i: (0, 0)),
            pl.BlockSpec((1, _NF), lambda i: (0, 0)),
        ],
        out_specs=pl.BlockSpec((blk, _NF), lambda i: (i, 0)),
        out_shape=jax.ShapeDtypeStruct((_N_PAIRS, _NF), jnp.float32),
    )(f_ij, rcut2d, w_filt_t, b_filt2d)


def _mul_rows(rows_ref, wij_ref):
    def _mrow(i, c2):
        for l in range(8):
            s = pl.ds(l * 16, 16)
            rows_ref[i, s] = rows_ref[i, s] * wij_ref[i, s]
        return c2

    lax.fori_loop(0, _C, _mrow, 0)


def _sc_edge_body(h_hbm, wij_hbm, idxi_hbm, idxj_hbm, out_hbm,
                  rows0, rows1, rows2, wij0, wij1, wij2,
                  ii0, ii1, ii2, ii3, ij0, ij1, ij2, ij3,
                  gs0, gs1, gs2, ws0, ws1, ws2, ss0, ss1, ss2,
                  is0, is1, is2, is3, agg_sh):
    cid = lax.axis_index("c")
    sid = lax.axis_index("s")
    wid = cid * _NS + sid

    rows = [rows0, rows1, rows2]
    wijb = [wij0, wij1, wij2]
    idxi = [ii0, ii1, ii2, ii3]
    idxj = [ij0, ij1, ij2, ij3]
    gsem = [gs0, gs1, gs2]
    wsem = [ws0, ws1, ws2]
    ssem = [ss0, ss1, ss2]
    isem = [is0, is1, is2, is3]

    z16 = jnp.zeros((16,), jnp.float32)

    def _zb(i, carry):
        r = i // 8
        c = (i % 8) * 16
        rows0[r, pl.ds(c, 16)] = z16
        return carry

    lax.fori_loop(0, _C * 8, _zb, 0)
    base_rows = sid * _ROWS_PER_TILE
    for k in range(_ROWS_PER_TILE // _C):
        pltpu.sync_copy(rows0, agg_sh.at[pl.ds(base_rows + k * _C, _C)])
    rem = _ROWS_PER_TILE % _C
    pltpu.sync_copy(rows0.at[pl.ds(0, rem)],
                    agg_sh.at[pl.ds(base_rows + _ROWS_PER_TILE - rem, rem)])

    @pl.when(sid == _NS - 1)
    def _zero_tail():
        pltpu.sync_copy(rows0.at[pl.ds(0, _N_ATOMS - _NS * _ROWS_PER_TILE)],
                        agg_sh.at[pl.ds(_NS * _ROWS_PER_TILE,
                                        _N_ATOMS - _NS * _ROWS_PER_TILE)])

    plsc.subcore_barrier()

    start = wid * _CPW

    def _fire_idx(c, pc):
        m = pc % 4
        pltpu.async_copy(idxi_hbm.at[pl.ds((start + c) * _C, _C)],
                         idxi[m], isem[m])
        pltpu.async_copy(idxj_hbm.at[pl.ds((start + c) * _C, _C)],
                         idxj[m], isem[m])

    def _wait_idx(c, pc):
        m = pc % 4
        pltpu.make_async_copy(idxi_hbm.at[pl.ds((start + c) * _C, _C)],
                              idxi[m], isem[m]).wait()
        pltpu.make_async_copy(idxj_hbm.at[pl.ds((start + c) * _C, _C)],
                              idxj[m], isem[m]).wait()

    def _fire_fetch(c, pc):
        k = pc % 3
        pltpu.async_copy(h_hbm.at[idxj[pc % 4]], rows[k], gsem[k])
        pltpu.async_copy(wij_hbm.at[pl.ds((start + c) * _C, _C)],
                         wijb[k], wsem[k])

    def _wait_fetch(c, pc):
        k = pc % 3
        pltpu.make_async_copy(h_hbm.at[idxj[pc % 4]], rows[k], gsem[k]).wait()
        pltpu.make_async_copy(wij_hbm.at[pl.ds((start + c) * _C, _C)],
                              wijb[k], wsem[k]).wait()

    def _fire_scatter(c, pc):
        k = pc % 3
        pltpu.async_copy(rows[k], agg_sh.at[idxi[pc % 4]], ssem[k], add=True)

    def _wait_scatter(c, pc):
        k = pc % 3
        pltpu.make_async_copy(rows[k], agg_sh.at[idxi[pc % 4]],
                              ssem[k]).wait()

    _fire_idx(0, 0)
    _fire_idx(1, 1)
    _wait_idx(0, 0)
    _fire_fetch(0, 0)

    def _iter(t, carry):
        for j in range(_U):
            c = t * _U + j
            if j >= 2:
                _wait_scatter(c - 2, j - 2)
            else:
                @pl.when(t >= 1)
                def _drain():
                    _wait_scatter(c - 2, j - 2)
            _fire_idx(c + 2, j + 2)
            _wait_idx(c + 1, j + 1)
            _fire_fetch(c + 1, j + 1)
            _wait_fetch(c, j)
            _mul_rows(rows[j % 3], wijb[j % 3])
            _fire_scatter(c, j)
        return carry

    lax.fori_loop(0, _CPW // _U, _iter, 0)

    _wait_scatter(_CPW - 2, _CPW - 2)
    _wait_scatter(_CPW - 1, _CPW - 1)
    _wait_fetch(_CPW, _CPW)
    _wait_idx(_CPW + 1, _CPW + 1)

    @pl.when(wid < _NTAIL)
    def _tail():
        ct = _NW * _CPW + wid
        pltpu.sync_copy(idxi_hbm.at[pl.ds(ct * _C, _C)], ii0)
        pltpu.sync_copy(idxj_hbm.at[pl.ds(ct * _C, _C)], ij0)
        pltpu.async_copy(h_hbm.at[ij0], rows0, gs0).wait()
        pltpu.sync_copy(wij_hbm.at[pl.ds(ct * _C, _C)], wij0)
        _mul_rows(rows0, wij0)
        pltpu.async_copy(rows0, agg_sh.at[ii0], ss0, add=True).wait()

    plsc.subcore_barrier()

    pltpu.sync_copy(agg_sh.at[pl.ds(base_rows, _ROWS_PER_TILE)],
                    out_hbm.at[cid, pl.ds(base_rows, _ROWS_PER_TILE)])

    @pl.when(sid == _NS - 1)
    def _write_tail():
        tail = _N_ATOMS - _NS * _ROWS_PER_TILE
        pltpu.sync_copy(agg_sh.at[pl.ds(_NS * _ROWS_PER_TILE, tail)],
                        out_hbm.at[cid, pl.ds(_NS * _ROWS_PER_TILE, tail)])


def _sc_edge(h, wij, idx_i, idx_j):
    mesh = plsc.VectorSubcoreMesh(core_axis_name="c", subcore_axis_name="s")
    f = functools.partial(
        pl.kernel,
        mesh=mesh,
        out_type=jax.ShapeDtypeStruct((_NC, _N_ATOMS, _NF), jnp.float32),
        scratch_types=(
            [pltpu.VMEM((_C, _NF), jnp.float32) for _ in range(6)]
            + [pltpu.VMEM((_C,), jnp.int32) for _ in range(8)]
            + [pltpu.SemaphoreType.DMA for _ in range(13)]
            + [pltpu.VMEM_SHARED((_N_ATOMS, _NF), jnp.float32)]
        ),
    )(_sc_edge_body)
    return f(h, wij, idx_i, idx_j)


def _out_body(p_ref, w_ref, b_ref, o_ref):
    a = p_ref[0] + p_ref[1]
    t = jnp.dot(a, w_ref[...], preferred_element_type=jnp.float32) + b_ref[...]
    o_ref[...] = _shifted_softplus(t)


def _compute_out(partials, w_out_t, b_out2d):
    blk = 2000
    grid = _N_ATOMS // blk
    return pl.pallas_call(
        _out_body,
        grid=(grid,),
        in_specs=[
            pl.BlockSpec((2, blk, _NF), lambda i: (0, i, 0)),
            pl.BlockSpec((_NF, _NF), lambda i: (0, 0)),
            pl.BlockSpec((1, _NF), lambda i: (0, 0)),
        ],
        out_specs=pl.BlockSpec((blk, _NF), lambda i: (i, 0)),
        out_shape=jax.ShapeDtypeStruct((_N_ATOMS, _NF), jnp.float32),
    )(partials, w_out_t, b_out2d)


def kernel(x, f_ij, idx_i, idx_j, rcut_ij, W_in, b_in, W_filt, b_filt,
           W_out, b_out):
    batch, n_atoms = x.shape[0], x.shape[1]
    x2d = x.reshape(batch * n_atoms, _NF)
    h = _compute_h(x2d, W_in.T, b_in.reshape(1, _NF))
    wij = _compute_wij(f_ij, rcut_ij.reshape(_N_PAIRS, 1), W_filt.T,
                       b_filt.reshape(1, _NF))
    partials = _sc_edge(h, wij, idx_i.astype(jnp.int32),
                        idx_j.astype(jnp.int32))
    out = _compute_out(partials, W_out.T, b_out.reshape(1, _NF))
    return out.reshape(batch, n_atoms, _NF)


# speedup vs baseline: 5.2926x; 1.7452x over previous
import functools

import jax
import jax.numpy as jnp
from jax import lax
from jax.experimental import pallas as pl
from jax.experimental.pallas import tpu as pltpu
from jax.experimental.pallas import tpu_sc as plsc

_LOG2 = 0.6931471805599453

_N_ATOMS = 10000
_N_PAIRS = 320000
_NF = 128

_NC = 2
_NS = 16
_NW = _NC * _NS
_C = 64
_NCHUNK = _N_PAIRS // _C
_CPW = _NCHUNK // _NW
_NTAIL = _NCHUNK - _CPW * _NW
_U = 12
_ROWS_PER_TILE = 624


def _shifted_softplus(t):
    return jnp.maximum(t, 0.0) + jnp.log1p(jnp.exp(-jnp.abs(t))) - _LOG2


def _h_body(x_ref, w_ref, b_ref, o_ref):
    o_ref[...] = (
        jnp.dot(x_ref[...], w_ref[...], preferred_element_type=jnp.float32)
        + b_ref[...]
    )


def _compute_h(x2d, w_in_t, b_in2d):
    blk = 2000
    grid = _N_ATOMS // blk
    return pl.pallas_call(
        _h_body,
        grid=(grid,),
        in_specs=[
            pl.BlockSpec((blk, _NF), lambda i: (i, 0)),
            pl.BlockSpec((_NF, _NF), lambda i: (0, 0)),
            pl.BlockSpec((1, _NF), lambda i: (0, 0)),
        ],
        out_specs=pl.BlockSpec((blk, _NF), lambda i: (i, 0)),
        out_shape=jax.ShapeDtypeStruct((_N_ATOMS, _NF), jnp.float32),
    )(x2d, w_in_t, b_in2d)


def _wij_body(ft_ref, w_ref, b_ref, o_ref):
    t = lax.dot_general(
        ft_ref[...], w_ref[...],
        dimension_numbers=(((0,), (0,)), ((), ())),
        preferred_element_type=jnp.float32,
    ) + b_ref[...]
    o_ref[...] = _shifted_softplus(t)


def _compute_wij(f_ij_t, w_filt_t, b_filt2d):
    blk = 3200
    grid = _N_PAIRS // blk
    n_rbf = f_ij_t.shape[0]
    return pl.pallas_call(
        _wij_body,
        grid=(grid,),
        in_specs=[
            pl.BlockSpec((n_rbf, blk), lambda i: (0, i)),
            pl.BlockSpec((n_rbf, _NF), lambda i: (0, 0)),
            pl.BlockSpec((1, _NF), lambda i: (0, 0)),
        ],
        out_specs=pl.BlockSpec((blk, _NF), lambda i: (i, 0)),
        out_shape=jax.ShapeDtypeStruct((_N_PAIRS, _NF), jnp.float32),
    )(f_ij_t, w_filt_t, b_filt2d)


def _mul_rows(rows_ref, wij_ref, rc_ref):
    def _mrow(i, c2):
        rc = rc_ref[pl.ds(i, 16)][0]
        for l in range(8):
            s = pl.ds(l * 16, 16)
            rows_ref[i, s] = rows_ref[i, s] * (wij_ref[i, s] * rc)
        return c2

    lax.fori_loop(0, _C, _mrow, 0)


def _sc_edge_body(h_hbm, wij_hbm, idxi_hbm, idxj_hbm, rcut_hbm, out_hbm,
                  rows0, rows1, rows2, wij0, wij1, wij2,
                  ii0, ii1, ii2, ii3, ij0, ij1, ij2, ij3,
                  rc0, rc1, rc2, rc3,
                  gs0, gs1, gs2, ws0, ws1, ws2, ss0, ss1, ss2,
                  is0, is1, is2, is3, agg_sh):
    cid = lax.axis_index("c")
    sid = lax.axis_index("s")
    wid = cid * _NS + sid

    rows = [rows0, rows1, rows2]
    wijb = [wij0, wij1, wij2]
    idxi = [ii0, ii1, ii2, ii3]
    idxj = [ij0, ij1, ij2, ij3]
    rcb = [rc0, rc1, rc2, rc3]
    gsem = [gs0, gs1, gs2]
    wsem = [ws0, ws1, ws2]
    ssem = [ss0, ss1, ss2]
    isem = [is0, is1, is2, is3]

    z16 = jnp.zeros((16,), jnp.float32)

    def _zb(i, carry):
        r = i // 8
        c = (i % 8) * 16
        rows0[r, pl.ds(c, 16)] = z16
        return carry

    lax.fori_loop(0, _C * 8, _zb, 0)
    base_rows = sid * _ROWS_PER_TILE
    for k in range(_ROWS_PER_TILE // _C):
        pltpu.sync_copy(rows0, agg_sh.at[pl.ds(base_rows + k * _C, _C)])
    rem = _ROWS_PER_TILE % _C
    pltpu.sync_copy(rows0.at[pl.ds(0, rem)],
                    agg_sh.at[pl.ds(base_rows + _ROWS_PER_TILE - rem, rem)])

    @pl.when(sid == _NS - 1)
    def _zero_tail():
        pltpu.sync_copy(rows0.at[pl.ds(0, _N_ATOMS - _NS * _ROWS_PER_TILE)],
                        agg_sh.at[pl.ds(_NS * _ROWS_PER_TILE,
                                        _N_ATOMS - _NS * _ROWS_PER_TILE)])

    plsc.subcore_barrier()

    start = wid * _CPW

    def _fire_idx(c, pc):
        m = pc % 4
        pltpu.async_copy(idxi_hbm.at[pl.ds((start + c) * _C, _C)],
                         idxi[m], isem[m])
        pltpu.async_copy(idxj_hbm.at[pl.ds((start + c) * _C, _C)],
                         idxj[m], isem[m])
        pltpu.async_copy(rcut_hbm.at[pl.ds((start + c) * _C, _C)],
                         rcb[m].at[pl.ds(0, _C)], isem[m])

    def _wait_idx(c, pc):
        m = pc % 4
        pltpu.make_async_copy(idxi_hbm.at[pl.ds((start + c) * _C, _C)],
                              idxi[m], isem[m]).wait()
        pltpu.make_async_copy(idxj_hbm.at[pl.ds((start + c) * _C, _C)],
                              idxj[m], isem[m]).wait()
        pltpu.make_async_copy(rcut_hbm.at[pl.ds((start + c) * _C, _C)],
                              rcb[m].at[pl.ds(0, _C)], isem[m]).wait()

    def _fire_fetch(c, pc):
        k = pc % 3
        pltpu.async_copy(h_hbm.at[idxj[pc % 4]], rows[k], gsem[k])
        pltpu.async_copy(wij_hbm.at[pl.ds((start + c) * _C, _C)],
                         wijb[k], wsem[k])

    def _wait_fetch(c, pc):
        k = pc % 3
        pltpu.make_async_copy(h_hbm.at[idxj[pc % 4]], rows[k], gsem[k]).wait()
        pltpu.make_async_copy(wij_hbm.at[pl.ds((start + c) * _C, _C)],
                              wijb[k], wsem[k]).wait()

    def _fire_scatter(c, pc):
        k = pc % 3
        pltpu.async_copy(rows[k], agg_sh.at[idxi[pc % 4]], ssem[k], add=True)

    def _wait_scatter(c, pc):
        k = pc % 3
        pltpu.make_async_copy(rows[k], agg_sh.at[idxi[pc % 4]],
                              ssem[k]).wait()

    _fire_idx(0, 0)
    _fire_idx(1, 1)
    _wait_idx(0, 0)
    _fire_fetch(0, 0)

    def _iter(t, carry):
        for j in range(_U):
            c = t * _U + j
            if j >= 2:
                _wait_scatter(c - 2, j - 2)
            else:
                @pl.when(t >= 1)
                def _drain():
                    _wait_scatter(c - 2, j - 2)
            _fire_idx(c + 2, j + 2)
            _wait_idx(c + 1, j + 1)
            _fire_fetch(c + 1, j + 1)
            _wait_fetch(c, j)
            _mul_rows(rows[j % 3], wijb[j % 3], rcb[j % 4])
            _fire_scatter(c, j)
        return carry

    lax.fori_loop(0, _CPW // _U, _iter, 0)

    _wait_scatter(_CPW - 2, _CPW - 2)
    _wait_scatter(_CPW - 1, _CPW - 1)
    _wait_fetch(_CPW, _CPW)
    _wait_idx(_CPW + 1, _CPW + 1)

    @pl.when(wid < _NTAIL)
    def _tail():
        ct = _NW * _CPW + wid
        pltpu.sync_copy(idxi_hbm.at[pl.ds(ct * _C, _C)], ii0)
        pltpu.sync_copy(idxj_hbm.at[pl.ds(ct * _C, _C)], ij0)
        pltpu.sync_copy(rcut_hbm.at[pl.ds(ct * _C, _C)],
                        rc0.at[pl.ds(0, _C)])
        pltpu.async_copy(h_hbm.at[ij0], rows0, gs0).wait()
        pltpu.sync_copy(wij_hbm.at[pl.ds(ct * _C, _C)], wij0)
        _mul_rows(rows0, wij0, rc0)
        pltpu.async_copy(rows0, agg_sh.at[ii0], ss0, add=True).wait()

    plsc.subcore_barrier()

    pltpu.sync_copy(agg_sh.at[pl.ds(base_rows, _ROWS_PER_TILE)],
                    out_hbm.at[cid, pl.ds(base_rows, _ROWS_PER_TILE)])

    @pl.when(sid == _NS - 1)
    def _write_tail():
        tail = _N_ATOMS - _NS * _ROWS_PER_TILE
        pltpu.sync_copy(agg_sh.at[pl.ds(_NS * _ROWS_PER_TILE, tail)],
                        out_hbm.at[cid, pl.ds(_NS * _ROWS_PER_TILE, tail)])


def _sc_edge(h, wij, idx_i, idx_j, rcut):
    mesh = plsc.VectorSubcoreMesh(core_axis_name="c", subcore_axis_name="s")
    f = functools.partial(
        pl.kernel,
        mesh=mesh,
        out_type=jax.ShapeDtypeStruct((_NC, _N_ATOMS, _NF), jnp.float32),
        scratch_types=(
            [pltpu.VMEM((_C, _NF), jnp.float32) for _ in range(6)]
            + [pltpu.VMEM((_C,), jnp.int32) for _ in range(8)]
            + [pltpu.VMEM((_C + 16,), jnp.float32) for _ in range(4)]
            + [pltpu.SemaphoreType.DMA for _ in range(13)]
            + [pltpu.VMEM_SHARED((_N_ATOMS, _NF), jnp.float32)]
        ),
    )(_sc_edge_body)
    return f(h, wij, idx_i, idx_j, rcut)


def _out_body(p_ref, w_ref, b_ref, o_ref):
    a = p_ref[0] + p_ref[1]
    t = jnp.dot(a, w_ref[...], preferred_element_type=jnp.float32) + b_ref[...]
    o_ref[...] = _shifted_softplus(t)


def _compute_out(partials, w_out_t, b_out2d):
    blk = 2000
    grid = _N_ATOMS // blk
    return pl.pallas_call(
        _out_body,
        grid=(grid,),
        in_specs=[
            pl.BlockSpec((2, blk, _NF), lambda i: (0, i, 0)),
            pl.BlockSpec((_NF, _NF), lambda i: (0, 0)),
            pl.BlockSpec((1, _NF), lambda i: (0, 0)),
        ],
        out_specs=pl.BlockSpec((blk, _NF), lambda i: (i, 0)),
        out_shape=jax.ShapeDtypeStruct((_N_ATOMS, _NF), jnp.float32),
    )(partials, w_out_t, b_out2d)


def kernel(x, f_ij, idx_i, idx_j, rcut_ij, W_in, b_in, W_filt, b_filt,
           W_out, b_out):
    batch, n_atoms = x.shape[0], x.shape[1]
    x2d = x.reshape(batch * n_atoms, _NF)
    h = _compute_h(x2d, W_in.T, b_in.reshape(1, _NF))
    wij = _compute_wij(f_ij.T, W_filt.T, b_filt.reshape(1, _NF))
    partials = _sc_edge(h, wij, idx_i.astype(jnp.int32),
                        idx_j.astype(jnp.int32), rcut_ij)
    out = _compute_out(partials, W_out.T, b_out.reshape(1, _NF))
    return out.reshape(batch, n_atoms, _NF)
